# baseline (speedup 1.0000x reference)
import functools
import math

import jax
import jax.numpy as jnp
from jax import lax
from jax.experimental import pallas as pl
from jax.experimental.pallas import tpu as pltpu
from jax.experimental.pallas import tpu_sc as plsc

_CH = 8
_RING = 3
_UNROLL = 2
_DO_COMPUTE = True


def _make_sc_kernel(B, S, D, S_COVER):
    info = plsc.get_sparse_core_info()
    NC, NS = info.num_cores, info.num_subcores
    NW = NC * NS
    rows_w = S_COVER // NW
    n_chunks = rows_w // _CH
    scale = math.sqrt(D)
    mesh = plsc.VectorSubcoreMesh(core_axis_name="c", subcore_axis_name="s")

    n_buf_per_set = 1 + B
    scratch = (
        [pltpu.VMEM((_CH, D), jnp.float32)] * (_RING * n_buf_per_set)
        + [pltpu.SemaphoreType.DMA] * (2 * _RING)
    )

    @functools.partial(
        pl.kernel, mesh=mesh,
        out_type=jax.ShapeDtypeStruct((B, S, D), jnp.float32),
        scratch_types=scratch,
    )
    def k(x_hbm, pos_hbm, out_hbm, *bufs):
        sets = []
        for r in range(_RING):
            base = r * n_buf_per_set
            sets.append({
                "pb": bufs[base],
                "xb": bufs[base + 1:base + 1 + B],
            })
        nb = _RING * n_buf_per_set
        sem_in = bufs[nb:nb + _RING]
        sem_out = bufs[nb + _RING:nb + 2 * _RING]

        wid = lax.axis_index("s") * NC + lax.axis_index("c")
        s0 = wid * rows_w

        def load_copies(c, r):
            row0 = s0 + c * _CH
            cps = [pltpu.make_async_copy(
                pos_hbm.at[pl.ds(row0, _CH), :], sets[r]["pb"], sem_in[r])]
            for b in range(B):
                cps.append(pltpu.make_async_copy(
                    x_hbm.at[b, pl.ds(row0, _CH), :],
                    sets[r]["xb"][b], sem_in[r]))
            return cps

        def store_copies(c, r):
            row0 = s0 + c * _CH
            return [pltpu.make_async_copy(
                sets[r]["xb"][b],
                out_hbm.at[b, pl.ds(row0, _CH), :],
                sem_out[r]) for b in range(B)]

        def compute(r):
            pb = sets[r]["pb"]
            xb = sets[r]["xb"]

            @plsc.parallel_loop(0, D, step=16, unroll=_UNROLL)
            def vec_body(i):
                for row in range(_CH):
                    p = pb[row, pl.ds(i, 16)]
                    for b in range(B):
                        xb[b][row, pl.ds(i, 16)] = (
                            xb[b][row, pl.ds(i, 16)] * scale + p)

        def slot_body(c, r, first, fire_more):
            for cp in load_copies(c, r):
                cp.wait()
            if _DO_COMPUTE:
                compute(r)
            for cp in store_copies(c, r):
                cp.start()
            if not first:
                rprev = (r - 1) % _RING
                for cp in store_copies(c - 1, rprev):
                    cp.wait()
                if fire_more:
                    def _fire():
                        for cp in load_copies(c + 2, (r + 2) % _RING):
                            cp.start()
                    if isinstance(c, int):
                        if c + 2 < n_chunks:
                            _fire()
                    else:
                        pl.when(c + 2 < n_chunks)(_fire)

        for cp in load_copies(0, 0) + load_copies(1, 1):
            cp.start()

        for cp in load_copies(0, 0):
            cp.wait()
        if _DO_COMPUTE:
            compute(0)
        for cp in store_copies(0, 0):
            cp.start()
        for cp in load_copies(2, 2):
            cp.start()

        n_grp = (n_chunks - 1) // _RING

        def group(g, carry):
            for j in range(_RING):
                c = 1 + _RING * g + j
                slot_body(c, (1 + j) % _RING, False, True)
            return carry

        lax.fori_loop(0, n_grp, group, 0)

        for c in range(1 + _RING * n_grp, n_chunks):
            slot_body(c, c % _RING, False, c + 2 < n_chunks)

        for cp in store_copies(n_chunks - 1, (n_chunks - 1) % _RING):
            cp.wait()

    return k


def _tc_body(x_ref, pos_ref, out_ref, *, scale):
    out_ref[...] = x_ref[...] * scale + pos_ref[...][None, :, :]


def _tc_call(x, pos_table, B, S, D):
    scale = math.sqrt(D)
    BS = 2048
    n_seq = S // BS
    grid = (n_seq, B)
    return pl.pallas_call(
        functools.partial(_tc_body, scale=scale),
        grid=grid,
        in_specs=[
            pl.BlockSpec((1, BS, D), lambda i, b: (b, i, 0)),
            pl.BlockSpec((BS, D), lambda i, b: (i, 0)),
        ],
        out_specs=pl.BlockSpec((1, BS, D), lambda i, b: (b, i, 0)),
        out_shape=jax.ShapeDtypeStruct((B, S, D), x.dtype),
    )(x, pos_table[:S])


def kernel(x, pos_table):
    B, S, D = x.shape
    k = _make_sc_kernel(B, S, D, S // 2)
    sc_out = k(x, pos_table[:S])
    tc_out = _tc_call(x, pos_table, B, S, D)
    return tc_out.at[0, 0, 0].add(sc_out[0, 0, 0] * 0.0)


# speedup vs baseline: 1.1741x; 1.1741x over previous
import functools
import math

import jax
import jax.numpy as jnp
from jax import lax
from jax.experimental import pallas as pl
from jax.experimental.pallas import tpu as pltpu
from jax.experimental.pallas import tpu_sc as plsc

_CH = 8
_RING = 3
_UNROLL = 4


def _make_sc_kernel(B, S, D):
    info = plsc.get_sparse_core_info()
    NC, NS = info.num_cores, info.num_subcores
    NW = NC * NS
    rows_w = S // NW
    n_chunks = rows_w // _CH
    scale = math.sqrt(D)
    mesh = plsc.VectorSubcoreMesh(core_axis_name="c", subcore_axis_name="s")

    n_buf_per_set = 1 + B
    scratch = (
        [pltpu.VMEM((_CH, D), jnp.float32)] * (_RING * n_buf_per_set)
        + [pltpu.SemaphoreType.DMA] * (2 * _RING)
    )

    @functools.partial(
        pl.kernel, mesh=mesh,
        out_type=jax.ShapeDtypeStruct((B, S, D), jnp.float32),
        scratch_types=scratch,
    )
    def k(x_hbm, pos_hbm, out_hbm, *bufs):
        sets = []
        for r in range(_RING):
            base = r * n_buf_per_set
            sets.append({
                "pb": bufs[base],
                "xb": bufs[base + 1:base + 1 + B],
            })
        nb = _RING * n_buf_per_set
        sem_in = bufs[nb:nb + _RING]
        sem_out = bufs[nb + _RING:nb + 2 * _RING]

        wid = lax.axis_index("s") * NC + lax.axis_index("c")
        s0 = wid * rows_w

        def load_copies(c, r):
            row0 = s0 + c * _CH
            cps = [pltpu.make_async_copy(
                pos_hbm.at[pl.ds(row0, _CH), :], sets[r]["pb"], sem_in[r])]
            for b in range(B):
                cps.append(pltpu.make_async_copy(
                    x_hbm.at[b, pl.ds(row0, _CH), :],
                    sets[r]["xb"][b], sem_in[r]))
            return cps

        def store_copies(c, r):
            row0 = s0 + c * _CH
            return [pltpu.make_async_copy(
                sets[r]["xb"][b],
                out_hbm.at[b, pl.ds(row0, _CH), :],
                sem_out[r]) for b in range(B)]

        def compute(r):
            pb = sets[r]["pb"]
            xb = sets[r]["xb"]

            @plsc.parallel_loop(0, D, step=16, unroll=_UNROLL)
            def vec_body(i):
                for row in range(_CH):
                    p = pb[row, pl.ds(i, 16)]
                    for b in range(B):
                        xb[b][row, pl.ds(i, 16)] = (
                            xb[b][row, pl.ds(i, 16)] * scale + p)

        def slot_body(c, r, fire_more):
            for cp in load_copies(c, r):
                cp.wait()
            compute(r)
            for cp in store_copies(c, r):
                cp.start()
            rprev = (r - 1) % _RING
            for cp in store_copies(c - 1, rprev):
                cp.wait()
            if fire_more:
                def _fire():
                    for cp in load_copies(c + 2, (r + 2) % _RING):
                        cp.start()
                if isinstance(c, int):
                    if c + 2 < n_chunks:
                        _fire()
                else:
                    pl.when(c + 2 < n_chunks)(_fire)

        for cp in load_copies(0, 0) + load_copies(1, 1):
            cp.start()

        for cp in load_copies(0, 0):
            cp.wait()
        compute(0)
        for cp in store_copies(0, 0):
            cp.start()
        for cp in load_copies(2, 2):
            cp.start()

        n_grp = (n_chunks - 1) // _RING

        def group(g, carry):
            for j in range(_RING):
                c = 1 + _RING * g + j
                slot_body(c, (1 + j) % _RING, True)
            return carry

        lax.fori_loop(0, n_grp, group, 0)

        for c in range(1 + _RING * n_grp, n_chunks):
            slot_body(c, c % _RING, c + 2 < n_chunks)

        for cp in store_copies(n_chunks - 1, (n_chunks - 1) % _RING):
            cp.wait()

    return k


def kernel(x, pos_table):
    B, S, D = x.shape
    k = _make_sc_kernel(B, S, D)
    return k(x, pos_table[:S])


# speedup vs baseline: 1.2345x; 1.0515x over previous
import functools
import math

import jax
import jax.numpy as jnp
from jax import lax
from jax.experimental import pallas as pl
from jax.experimental.pallas import tpu as pltpu
from jax.experimental.pallas import tpu_sc as plsc

_CH = 8
_RING = 3
_UNROLL = 2


def _make_sc_kernel(B, S, D):
    info = plsc.get_sparse_core_info()
    NC, NS = info.num_cores, info.num_subcores
    NW = NC * NS
    rows_w = S // NW
    n_chunks = rows_w // _CH
    scale = math.sqrt(D)
    mesh = plsc.VectorSubcoreMesh(core_axis_name="c", subcore_axis_name="s")

    n_buf_per_set = 1 + B
    scratch = (
        [pltpu.VMEM((_CH, D), jnp.float32)] * (_RING * n_buf_per_set)
        + [pltpu.SemaphoreType.DMA] * (2 * _RING)
    )

    @functools.partial(
        pl.kernel, mesh=mesh,
        out_type=jax.ShapeDtypeStruct((B, S, D), jnp.float32),
        scratch_types=scratch,
    )
    def k(x_hbm, pos_hbm, out_hbm, *bufs):
        sets = []
        for r in range(_RING):
            base = r * n_buf_per_set
            sets.append({
                "pb": bufs[base],
                "xb": bufs[base + 1:base + 1 + B],
            })
        nb = _RING * n_buf_per_set
        sem_in = bufs[nb:nb + _RING]
        sem_out = bufs[nb + _RING:nb + 2 * _RING]

        wid = lax.axis_index("s") * NC + lax.axis_index("c")
        s0 = wid * rows_w

        def load_copies(c, r):
            row0 = s0 + c * _CH
            cps = [pltpu.make_async_copy(
                pos_hbm.at[pl.ds(row0, _CH), :], sets[r]["pb"], sem_in[r])]
            for b in range(B):
                cps.append(pltpu.make_async_copy(
                    x_hbm.at[b, pl.ds(row0, _CH), :],
                    sets[r]["xb"][b], sem_in[r]))
            return cps

        def store_copies(c, r):
            row0 = s0 + c * _CH
            return [pltpu.make_async_copy(
                sets[r]["xb"][b],
                out_hbm.at[b, pl.ds(row0, _CH), :],
                sem_out[r]) for b in range(B)]

        def compute(r):
            pb = sets[r]["pb"]
            xb = sets[r]["xb"]

            @plsc.parallel_loop(0, D, step=16, unroll=_UNROLL)
            def vec_body(i):
                for row in range(_CH):
                    p = pb[row, pl.ds(i, 16)]
                    for b in range(B):
                        xb[b][row, pl.ds(i, 16)] = (
                            xb[b][row, pl.ds(i, 16)] * scale + p)

        def slot_body(c, r, fire_more):
            for cp in load_copies(c, r):
                cp.wait()
            compute(r)
            for cp in store_copies(c, r):
                cp.start()
            rprev = (r - 1) % _RING
            for cp in store_copies(c - 1, rprev):
                cp.wait()
            if fire_more:
                def _fire():
                    for cp in load_copies(c + 2, (r + 2) % _RING):
                        cp.start()
                if isinstance(c, int):
                    if c + 2 < n_chunks:
                        _fire()
                else:
                    pl.when(c + 2 < n_chunks)(_fire)

        for cp in load_copies(0, 0) + load_copies(1, 1):
            cp.start()

        for cp in load_copies(0, 0):
            cp.wait()
        compute(0)
        for cp in store_copies(0, 0):
            cp.start()
        for cp in load_copies(2, 2):
            cp.start()

        n_grp = (n_chunks - 1) // _RING

        def group(g, carry):
            for j in range(_RING):
                c = 1 + _RING * g + j
                slot_body(c, (1 + j) % _RING, True)
            return carry

        lax.fori_loop(0, n_grp, group, 0)

        for c in range(1 + _RING * n_grp, n_chunks):
            slot_body(c, c % _RING, c + 2 < n_chunks)

        for cp in store_copies(n_chunks - 1, (n_chunks - 1) % _RING):
            cp.wait()

    return k


def kernel(x, pos_table):
    B, S, D = x.shape
    k = _make_sc_kernel(B, S, D)
    return k(x, pos_table[:S])


# speedup vs baseline: 1.2518x; 1.0140x over previous
import functools
import math

import jax
import jax.numpy as jnp
from jax import lax
from jax.experimental import pallas as pl
from jax.experimental.pallas import tpu as pltpu
from jax.experimental.pallas import tpu_sc as plsc

_CH = 8
_UNROLL = 2


def _make_sc_kernel(B, S, D):
    info = plsc.get_sparse_core_info()
    NC, NS = info.num_cores, info.num_subcores
    NW = NC * NS
    rows_w = S // NW
    n_chunks = rows_w // _CH
    scale = math.sqrt(D)
    mesh = plsc.VectorSubcoreMesh(core_axis_name="c", subcore_axis_name="s")

    n_buf_per_set = 1 + B
    scratch = [pltpu.VMEM((_CH, D), jnp.float32)] * (2 * n_buf_per_set) + [
        pltpu.SemaphoreType.DMA,
        pltpu.SemaphoreType.DMA,
        pltpu.SemaphoreType.DMA,
        pltpu.SemaphoreType.DMA,
    ]

    @functools.partial(
        pl.kernel, mesh=mesh,
        out_type=jax.ShapeDtypeStruct((B, S, D), jnp.float32),
        scratch_types=scratch,
    )
    def k(x_hbm, pos_hbm, out_hbm, *bufs):
        sets = []
        for r in range(2):
            base = r * n_buf_per_set
            sets.append({
                "pb": bufs[base],
                "xb": bufs[base + 1:base + 1 + B],
            })
        sem_in = bufs[2 * n_buf_per_set:2 * n_buf_per_set + 2]
        sem_out = bufs[2 * n_buf_per_set + 2:2 * n_buf_per_set + 4]

        wid = lax.axis_index("s") * NC + lax.axis_index("c")
        s0 = wid * rows_w

        def load_copies(c, r):
            row0 = s0 + c * _CH
            cps = [pltpu.make_async_copy(
                pos_hbm.at[pl.ds(row0, _CH), :], sets[r]["pb"], sem_in[r])]
            for b in range(B):
                cps.append(pltpu.make_async_copy(
                    x_hbm.at[b, pl.ds(row0, _CH), :],
                    sets[r]["xb"][b], sem_in[r]))
            return cps

        def store_copies(c, r):
            row0 = s0 + c * _CH
            return [pltpu.make_async_copy(
                sets[r]["xb"][b],
                out_hbm.at[b, pl.ds(row0, _CH), :],
                sem_out[r]) for b in range(B)]

        def compute(r):
            pb = sets[r]["pb"]
            xb = sets[r]["xb"]

            @plsc.parallel_loop(0, D, step=16, unroll=_UNROLL)
            def vec_body(i):
                for row in range(_CH):
                    p = pb[row, pl.ds(i, 16)]
                    for b in range(B):
                        xb[b][row, pl.ds(i, 16)] = (
                            xb[b][row, pl.ds(i, 16)] * scale + p)

        for cp in load_copies(0, 0) + load_copies(1, 1):
            cp.start()

        def slot_body(c, r):
            for cp in load_copies(c, r):
                cp.wait()
            compute(r)
            for cp in store_copies(c, r):
                cp.start()
            @pl.when(c + 2 < n_chunks)
            def _():
                for cp in store_copies(c, r):
                    cp.wait()
                for cp in load_copies(c + 2, r):
                    cp.start()

        def group(g, carry):
            for r in range(2):
                slot_body(2 * g + r, r)
            return carry

        lax.fori_loop(0, n_chunks // 2, group, 0)

        for r in range(2):
            for cp in store_copies(n_chunks - 2 + r, r):
                cp.wait()

    return k


def kernel(x, pos_table):
    B, S, D = x.shape
    k = _make_sc_kernel(B, S, D)
    return k(x, pos_table[:S])
